# Initial kernel scaffold; baseline (speedup 1.0000x reference)
import jax
import jax.numpy as jnp
from jax.experimental import pallas as pl


def kernel(flat, cu_seqlens):
    raise NotImplementedError("write your pallas kernel here")


# speedup vs baseline: 1.9305x; 1.9305x over previous
import functools

import jax
import jax.numpy as jnp
from jax import lax
from jax.experimental import pallas as pl
from jax.experimental.pallas import tpu as pltpu
from jax.experimental.pallas import tpu_sc as plsc

_TOTAL = 16384
_D = 2048
_NSEQ = 8
_BLOCK = 128
_MAXJ = 128
_NCHUNK = _MAXJ * _NSEQ
_NWORKERS = 32
_ROWS_PER_W = _TOTAL // _NWORKERS
_GROWS = 16
_STEPS = _ROWS_PER_W // _GROWS

_mesh = plsc.VectorSubcoreMesh(core_axis_name="c", subcore_axis_name="s")


@functools.partial(
    pl.kernel,
    mesh=_mesh,
    compiler_params=pltpu.CompilerParams(needs_layout_passes=False),
    out_type=jax.ShapeDtypeStruct((_TOTAL, _D), jnp.float32),
    scratch_types=[
        pltpu.VMEM((16,), jnp.int32),
        pltpu.VMEM((_NCHUNK,), jnp.int32),
        pltpu.VMEM((_GROWS, _D), jnp.float32),
        pltpu.VMEM((_GROWS, _D), jnp.float32),
        pltpu.SemaphoreType.DMA,
        pltpu.SemaphoreType.DMA,
        pltpu.SemaphoreType.DMA,
    ],
)
def _pack(cu_hbm, flat_hbm, out_hbm, cu_v, base_tab, rows0, rows1,
          gsem, ssem0, ssem1):
    wid = lax.axis_index("s") * 2 + lax.axis_index("c")
    pltpu.sync_copy(cu_hbm, cu_v)
    iota = lax.iota(jnp.int32, 16)

    def build(i, carry):
        c = i * 16 + iota
        jv = c >> 3
        sv = c & 7
        lo = plsc.load_gather(cu_v, [sv])
        hi = plsc.load_gather(cu_v, [sv + 1])
        ln = jnp.clip(hi - lo - (jv << 7), 0, _BLOCK)
        cs = jnp.cumsum(ln)
        base_tab[pl.ds(i * 16, 16)] = carry + cs - ln
        return carry + jnp.sum(ln)

    lax.fori_loop(0, _NCHUNK // 16, build, jnp.int32(0), unroll=4)

    def src_rows(it):
        obase = wid * _ROWS_PER_W + it * _GROWS
        o = obase + iota
        res = jnp.zeros((16,), jnp.int32)
        for step in (512, 256, 128, 64, 32, 16, 8, 4, 2, 1):
            cand = res + step
            v = plsc.load_gather(base_tab, [cand])
            res = jnp.where(v <= o, cand, res)
        b = plsc.load_gather(base_tab, [res])
        src = plsc.load_gather(cu_v, [res & 7]) + ((res >> 3) << 7) + (o - b)
        return src, obase

    def half(it, rows, ssem, is_first):
        @pl.when(jnp.logical_not(is_first))
        def _():
            pltpu.make_async_copy(
                rows, out_hbm.at[pl.ds(0, _GROWS)], ssem).wait()

        src, obase = src_rows(it)
        pltpu.async_copy(flat_hbm.at[src], rows, gsem).wait()
        pltpu.async_copy(rows, out_hbm.at[pl.ds(obase, _GROWS)], ssem)

    def body(i, carry):
        half(2 * i, rows0, ssem0, i == 0)
        half(2 * i + 1, rows1, ssem1, i == 0)
        return carry

    lax.fori_loop(0, _STEPS // 2, body, jnp.int32(0))
    pltpu.make_async_copy(rows0, out_hbm.at[pl.ds(0, _GROWS)], ssem0).wait()
    pltpu.make_async_copy(rows1, out_hbm.at[pl.ds(0, _GROWS)], ssem1).wait()


def kernel(flat, cu_seqlens):
    cu16 = jnp.zeros((16,), jnp.int32).at[: cu_seqlens.shape[0]].set(
        cu_seqlens.astype(jnp.int32))
    return _pack(cu16, flat)
